# Initial kernel scaffold; baseline (speedup 1.0000x reference)
import jax
import jax.numpy as jnp
from jax.experimental import pallas as pl


def kernel(ph_dur, ph2word, word_dur):
    raise NotImplementedError("write your pallas kernel here")


# speedup vs baseline: 5.0159x; 5.0159x over previous
import functools

import jax
import jax.numpy as jnp
from jax import lax
from jax.experimental import pallas as pl
from jax.experimental.pallas import tpu as pltpu, tpu_sc as plsc

B, T_PH, T_W = 16, 2048, 1024
EPS = 1e-05
L = 16
MAGIC = 12582912.0


def _body(ph_hbm, idx_hbm, wd_hbm, out_hbm,
          ph_v, idx_v, wd_v, seg_v, alpha_v, out_v, sem):
    row = lax.axis_index("s")
    half = lax.axis_index("c")

    cp_ph = pltpu.async_copy(ph_hbm.at[row], ph_v, sem)
    cp_ix = pltpu.async_copy(idx_hbm.at[row], idx_v, sem)
    cp_wd = pltpu.async_copy(wd_hbm.at[row], wd_v, sem)
    cp_ph.wait()
    cp_ix.wait()
    cp_wd.wait()

    zeros = jnp.zeros((L,), jnp.float32)

    def zero_step(i, _):
        seg_v[pl.ds(i * L, L)] = zeros
        return 0

    lax.fori_loop(0, T_W // L, zero_step, 0)

    def scat_step(i, _):
        idx = idx_v[pl.ds(i * L, L)]
        vals = ph_v[pl.ds(i * L, L)]
        mask = idx > 0
        plsc.addupdate_scatter(seg_v, [jnp.maximum(idx - 1, 0)], vals,
                               mask=mask)
        return 0

    lax.fori_loop(0, T_PH // L, scat_step, 0)

    def alpha_step(i, _):
        s = seg_v[pl.ds(i * L, L)]
        w = wd_v[pl.ds(i * L, L)]
        alpha_v[pl.ds(i * L, L)] = w / jnp.maximum(s, EPS)
        return 0

    lax.fori_loop(0, T_W // L, alpha_step, 0)

    base = half * (T_PH // 2)

    def gath_step(i, _):
        off = base + i * L
        idx = idx_v[pl.ds(off, L)]
        vals = ph_v[pl.ds(off, L)]
        mask = idx > 0
        a = plsc.load_gather(alpha_v, [jnp.maximum(idx - 1, 0)], mask=mask)
        x = jnp.where(mask, vals * a, 0.0)
        r = (x + MAGIC) - MAGIC
        out_v[pl.ds(i * L, L)] = r.astype(jnp.int32)
        return 0

    lax.fori_loop(0, T_PH // 2 // L, gath_step, 0)

    pltpu.sync_copy(out_v, out_hbm.at[row, pl.ds(base, T_PH // 2)])


@jax.jit
def _regulate(ph_dur, ph2word_i32, word_dur):
    mesh = plsc.VectorSubcoreMesh(core_axis_name="c", subcore_axis_name="s")
    f = functools.partial(
        pl.kernel,
        out_type=jax.ShapeDtypeStruct((B, T_PH), jnp.int32),
        mesh=mesh,
        compiler_params=pltpu.CompilerParams(needs_layout_passes=False),
        scratch_types=[
            pltpu.VMEM((T_PH,), jnp.float32),
            pltpu.VMEM((T_PH,), jnp.int32),
            pltpu.VMEM((T_W,), jnp.float32),
            pltpu.VMEM((T_W,), jnp.float32),
            pltpu.VMEM((T_W,), jnp.float32),
            pltpu.VMEM((T_PH // 2,), jnp.int32),
            pltpu.SemaphoreType.DMA,
        ],
    )(_body)
    return f(ph_dur, ph2word_i32, word_dur)


def kernel(ph_dur, ph2word, word_dur):
    out = _regulate(ph_dur.astype(jnp.float32), ph2word.astype(jnp.int32),
                    word_dur.astype(jnp.float32))
    return out.astype(jnp.int64)
